# baseline (speedup 1.0000x reference)
import jax
import jax.numpy as jnp
from jax.experimental import pallas as pl
from jax.experimental.pallas import tpu as pltpu

_TN = 1024


def _vq_kernel(xt_ref, embt_ref, hi_ref, lo_ref, e2r_ref, e2c_ref, f2r_ref,
               f2c_ref, dist_ref, indt_ref, qt_ref):
    xt = xt_ref[0]
    embt = embt_ref[...]
    xt2 = xt + xt
    cdim = (((0,), (0,)), ((), ()))
    fe2 = jax.lax.dot_general(
        xt2, embt, dimension_numbers=cdim,
        preferred_element_type=jnp.float32)
    dist_ref[0] = (fe2 - f2c_ref[0]) - e2r_ref[...]
    feT2 = jax.lax.dot_general(
        embt, xt2, dimension_numbers=cdim,
        preferred_element_type=jnp.float32)
    distT = (feT2 - f2r_ref[0]) - e2c_ref[...]
    indT = jnp.argmax(distT, axis=0)
    indt_ref[0, 0] = indT
    k_iota = jax.lax.broadcasted_iota(jnp.int32, distT.shape, 0)
    onehot = (k_iota == indT[None, :]).astype(jnp.float32)
    qt_ref[0] = (
        jax.lax.dot_general(hi_ref[...], onehot, dimension_numbers=cdim,
                            preferred_element_type=jnp.float32)
        + jax.lax.dot_general(lo_ref[...], onehot, dimension_numbers=cdim,
                              preferred_element_type=jnp.float32))


@jax.jit
def kernel(x, embed):
    H, K, D = embed.shape
    orig_shape = x.shape
    N = x.size // (H * D)
    G = N // _TN
    xT = x.reshape(G, _TN, D).transpose(0, 2, 1)
    emb2 = embed.reshape(K, D)
    embT = emb2.T
    emb_hi = emb2.astype(jnp.bfloat16).astype(jnp.float32)
    emb_lo = emb2 - emb_hi
    e2 = jnp.sum(embed ** 2, axis=-1)
    e2c = e2.reshape(K, 1)
    flatten = x.reshape(H, -1, D)
    f2 = jnp.sum(flatten ** 2, axis=-1)
    f2r = f2.reshape(G, 1, _TN)
    f2c = f2.reshape(G, _TN, 1)

    dist, indT, qT = pl.pallas_call(
        _vq_kernel,
        grid=(G,),
        in_specs=[
            pl.BlockSpec((1, D, _TN), lambda i: (i, 0, 0)),
            pl.BlockSpec((D, K), lambda i: (0, 0)),
            pl.BlockSpec((K, D), lambda i: (0, 0)),
            pl.BlockSpec((K, D), lambda i: (0, 0)),
            pl.BlockSpec((1, K), lambda i: (0, 0)),
            pl.BlockSpec((K, 1), lambda i: (0, 0)),
            pl.BlockSpec((1, 1, _TN), lambda i: (i, 0, 0)),
            pl.BlockSpec((1, _TN, 1), lambda i: (i, 0, 0)),
        ],
        out_specs=[
            pl.BlockSpec((1, _TN, K), lambda i: (i, 0, 0)),
            pl.BlockSpec((1, 1, _TN), lambda i: (i, 0, 0)),
            pl.BlockSpec((1, D, _TN), lambda i: (i, 0, 0)),
        ],
        out_shape=[
            jax.ShapeDtypeStruct((G, _TN, K), jnp.float32),
            jax.ShapeDtypeStruct((G, 1, _TN), jnp.int32),
            jax.ShapeDtypeStruct((G, D, _TN), jnp.float32),
        ],
        compiler_params=pltpu.CompilerParams(
            dimension_semantics=("arbitrary",)),
    )(xT, embT, emb_hi, emb_lo, e2, e2c, f2r, f2c)

    quantize = qT.transpose(0, 2, 1).reshape(orig_shape)
    return (quantize,
            indT.reshape(orig_shape[:-1]),
            dist.reshape(H, N, K))


# speedup vs baseline: 1.0131x; 1.0131x over previous
import jax
import jax.numpy as jnp
from jax.experimental import pallas as pl
from jax.experimental.pallas import tpu as pltpu

_TN = 1024


def _vq_kernel(xt_ref, embt_ref, hi_ref, lo_ref, e2r_ref, e2c_ref, f2r_ref,
               f2c_ref, dist_ref, indt_ref, qt_ref):
    xt = xt_ref[0]
    embt = embt_ref[...]
    xt2 = xt + xt
    cdim = (((0,), (0,)), ((), ()))
    fe2 = jax.lax.dot_general(
        xt2, embt, dimension_numbers=cdim,
        preferred_element_type=jnp.float32)
    dist_ref[0] = (fe2 - f2c_ref[0]) - e2r_ref[...]
    feT2 = jax.lax.dot_general(
        embt, xt2, dimension_numbers=cdim,
        preferred_element_type=jnp.float32)
    distT = (feT2 - f2r_ref[0]) - e2c_ref[...]
    indT = jnp.argmax(distT, axis=0)
    indt_ref[0, 0] = indT
    k_iota = jax.lax.broadcasted_iota(jnp.int32, distT.shape, 0)
    onehot = (k_iota == indT[None, :]).astype(jnp.float32)
    onehot_bf = onehot.astype(jnp.bfloat16)
    qt_ref[0] = (
        jax.lax.dot_general(hi_ref[...], onehot_bf, dimension_numbers=cdim,
                            preferred_element_type=jnp.float32)
        + jax.lax.dot_general(lo_ref[...], onehot_bf, dimension_numbers=cdim,
                              preferred_element_type=jnp.float32))


@jax.jit
def kernel(x, embed):
    H, K, D = embed.shape
    orig_shape = x.shape
    N = x.size // (H * D)
    G = N // _TN
    xT = x.reshape(G, _TN, D).transpose(0, 2, 1)
    emb2 = embed.reshape(K, D)
    embT = emb2.T
    emb_hi = emb2.astype(jnp.bfloat16)
    emb_lo = (emb2 - emb_hi.astype(jnp.float32)).astype(jnp.bfloat16)
    e2 = jnp.sum(embed ** 2, axis=-1)
    e2c = e2.reshape(K, 1)
    flatten = x.reshape(H, -1, D)
    f2 = jnp.sum(flatten ** 2, axis=-1)
    f2r = f2.reshape(G, 1, _TN)
    f2c = f2.reshape(G, _TN, 1)

    dist, indT, qT = pl.pallas_call(
        _vq_kernel,
        grid=(G,),
        in_specs=[
            pl.BlockSpec((1, D, _TN), lambda i: (i, 0, 0)),
            pl.BlockSpec((D, K), lambda i: (0, 0)),
            pl.BlockSpec((K, D), lambda i: (0, 0)),
            pl.BlockSpec((K, D), lambda i: (0, 0)),
            pl.BlockSpec((1, K), lambda i: (0, 0)),
            pl.BlockSpec((K, 1), lambda i: (0, 0)),
            pl.BlockSpec((1, 1, _TN), lambda i: (i, 0, 0)),
            pl.BlockSpec((1, _TN, 1), lambda i: (i, 0, 0)),
        ],
        out_specs=[
            pl.BlockSpec((1, _TN, K), lambda i: (i, 0, 0)),
            pl.BlockSpec((1, 1, _TN), lambda i: (i, 0, 0)),
            pl.BlockSpec((1, D, _TN), lambda i: (i, 0, 0)),
        ],
        out_shape=[
            jax.ShapeDtypeStruct((G, _TN, K), jnp.float32),
            jax.ShapeDtypeStruct((G, 1, _TN), jnp.int32),
            jax.ShapeDtypeStruct((G, D, _TN), jnp.float32),
        ],
        compiler_params=pltpu.CompilerParams(
            dimension_semantics=("arbitrary",)),
    )(xT, embT, emb_hi, emb_lo, e2, e2c, f2r, f2c)

    quantize = qT.transpose(0, 2, 1).reshape(orig_shape)
    return (quantize,
            indT.reshape(orig_shape[:-1]),
            dist.reshape(H, N, K))


# speedup vs baseline: 1.1742x; 1.1590x over previous
import jax
import jax.numpy as jnp
from jax.experimental import pallas as pl
from jax.experimental.pallas import tpu as pltpu

_TN = 1024


def _vq_kernel(xt_ref, embt_ref, hilo_ref, e2r_ref, e2c_ref, f2r_ref,
               f2c_ref, dist_ref, indt_ref, qt_ref):
    xt = xt_ref[0]
    embt = embt_ref[...]
    xt2 = xt + xt
    cdim = (((0,), (0,)), ((), ()))
    fe2 = jax.lax.dot_general(
        xt2, embt, dimension_numbers=cdim,
        preferred_element_type=jnp.float32)
    dist_ref[0] = (fe2 - f2c_ref[0]) - e2r_ref[...]
    feT2 = jax.lax.dot_general(
        embt, xt2, dimension_numbers=cdim,
        preferred_element_type=jnp.float32)
    distT = (feT2 - f2r_ref[0]) - e2c_ref[...]
    indT = jnp.argmax(distT, axis=0)
    indt_ref[0, 0] = indT
    k_iota = jax.lax.broadcasted_iota(jnp.int32, distT.shape, 0)
    onehot = (k_iota == indT[None, :]).astype(jnp.float32)
    onehot_bf = onehot.astype(jnp.bfloat16)
    qt2 = jax.lax.dot_general(
        hilo_ref[...], onehot_bf, dimension_numbers=cdim,
        preferred_element_type=jnp.float32)
    D = xt.shape[0]
    qt_ref[0] = qt2[:D] + qt2[D:]


@jax.jit
def kernel(x, embed):
    H, K, D = embed.shape
    orig_shape = x.shape
    N = x.size // (H * D)
    G = N // _TN
    xT = x.reshape(G, _TN, D).transpose(0, 2, 1)
    emb2 = embed.reshape(K, D)
    embT = emb2.T
    emb_hi = emb2.astype(jnp.bfloat16)
    hi32 = jax.lax.optimization_barrier(emb_hi).astype(jnp.float32)
    emb_lo = (emb2 - hi32).astype(jnp.bfloat16)
    hilo = jnp.concatenate([emb_hi, emb_lo], axis=1)
    e2 = jnp.sum(embed ** 2, axis=-1)
    e2c = e2.reshape(K, 1)
    flatten = x.reshape(H, -1, D)
    f2 = jnp.sum(flatten ** 2, axis=-1)
    f2r = f2.reshape(G, 1, _TN)
    f2c = f2.reshape(G, _TN, 1)

    dist, indT, qT = pl.pallas_call(
        _vq_kernel,
        grid=(G,),
        in_specs=[
            pl.BlockSpec((1, D, _TN), lambda i: (i, 0, 0)),
            pl.BlockSpec((D, K), lambda i: (0, 0)),
            pl.BlockSpec((K, 2 * D), lambda i: (0, 0)),
            pl.BlockSpec((1, K), lambda i: (0, 0)),
            pl.BlockSpec((K, 1), lambda i: (0, 0)),
            pl.BlockSpec((1, 1, _TN), lambda i: (i, 0, 0)),
            pl.BlockSpec((1, _TN, 1), lambda i: (i, 0, 0)),
        ],
        out_specs=[
            pl.BlockSpec((1, _TN, K), lambda i: (i, 0, 0)),
            pl.BlockSpec((1, 1, _TN), lambda i: (i, 0, 0)),
            pl.BlockSpec((1, D, _TN), lambda i: (i, 0, 0)),
        ],
        out_shape=[
            jax.ShapeDtypeStruct((G, _TN, K), jnp.float32),
            jax.ShapeDtypeStruct((G, 1, _TN), jnp.int32),
            jax.ShapeDtypeStruct((G, D, _TN), jnp.float32),
        ],
        compiler_params=pltpu.CompilerParams(
            dimension_semantics=("arbitrary",)),
    )(xT, embT, hilo, e2, e2c, f2r, f2c)

    quantize = qT.transpose(0, 2, 1).reshape(orig_shape)
    return (quantize,
            indT.reshape(orig_shape[:-1]),
            dist.reshape(H, N, K))


# speedup vs baseline: 1.2272x; 1.0452x over previous
import jax
import jax.numpy as jnp
from jax.experimental import pallas as pl
from jax.experimental.pallas import tpu as pltpu

_TN = 1024


def _vq_kernel(xt_ref, embt_ref, hilo_ref, e2r_ref, e2c_ref, f2r_ref,
               dist_ref, indt_ref, qt_ref):
    xt = xt_ref[0]
    embt = embt_ref[...]
    xt2 = xt + xt
    cdim = (((0,), (0,)), ((), ()))
    fe2 = jax.lax.dot_general(
        xt2, embt, dimension_numbers=cdim,
        preferred_element_type=jnp.float32)
    x2 = xt * xt
    ones = jnp.ones((xt.shape[0], 1), jnp.float32)
    f2c = jax.lax.dot_general(
        x2, ones, dimension_numbers=cdim,
        preferred_element_type=jnp.float32)
    dist_ref[0] = (fe2 - f2c) - e2r_ref[...]
    feT2 = jax.lax.dot_general(
        embt, xt2, dimension_numbers=cdim,
        preferred_element_type=jnp.float32)
    distT = (feT2 - f2r_ref[0]) - e2c_ref[...]
    indT = jnp.argmax(distT, axis=0)
    indt_ref[0, 0] = indT
    k_iota = jax.lax.broadcasted_iota(jnp.int32, distT.shape, 0)
    onehot = (k_iota == indT[None, :]).astype(jnp.float32)
    onehot_bf = onehot.astype(jnp.bfloat16)
    qt2 = jax.lax.dot_general(
        hilo_ref[...], onehot_bf, dimension_numbers=cdim,
        preferred_element_type=jnp.float32)
    D = xt.shape[0]
    qt_ref[0] = qt2[:D] + qt2[D:]


@jax.jit
def kernel(x, embed):
    H, K, D = embed.shape
    orig_shape = x.shape
    N = x.size // (H * D)
    G = N // _TN
    xT = x.reshape(G, _TN, D).transpose(0, 2, 1)
    emb2 = embed.reshape(K, D)
    embT = emb2.T
    emb_hi = emb2.astype(jnp.bfloat16)
    hi32 = jax.lax.optimization_barrier(emb_hi).astype(jnp.float32)
    emb_lo = (emb2 - hi32).astype(jnp.bfloat16)
    hilo = jnp.concatenate([emb_hi, emb_lo], axis=1)
    e2 = jnp.sum(embed ** 2, axis=-1)
    e2c = e2.reshape(K, 1)
    flatten = x.reshape(H, -1, D)
    f2 = jnp.sum(flatten ** 2, axis=-1)
    f2r = f2.reshape(G, 1, _TN)

    dist, indT, qT = pl.pallas_call(
        _vq_kernel,
        grid=(G,),
        in_specs=[
            pl.BlockSpec((1, D, _TN), lambda i: (i, 0, 0)),
            pl.BlockSpec((D, K), lambda i: (0, 0)),
            pl.BlockSpec((K, 2 * D), lambda i: (0, 0)),
            pl.BlockSpec((1, K), lambda i: (0, 0)),
            pl.BlockSpec((K, 1), lambda i: (0, 0)),
            pl.BlockSpec((1, 1, _TN), lambda i: (i, 0, 0)),
        ],
        out_specs=[
            pl.BlockSpec((1, _TN, K), lambda i: (i, 0, 0)),
            pl.BlockSpec((1, 1, _TN), lambda i: (i, 0, 0)),
            pl.BlockSpec((1, D, _TN), lambda i: (i, 0, 0)),
        ],
        out_shape=[
            jax.ShapeDtypeStruct((G, _TN, K), jnp.float32),
            jax.ShapeDtypeStruct((G, 1, _TN), jnp.int32),
            jax.ShapeDtypeStruct((G, D, _TN), jnp.float32),
        ],
        compiler_params=pltpu.CompilerParams(
            dimension_semantics=("arbitrary",)),
    )(xT, embT, hilo, e2, e2c, f2r)

    quantize = qT.transpose(0, 2, 1).reshape(orig_shape)
    return (quantize,
            indT.reshape(orig_shape[:-1]),
            dist.reshape(H, N, K))
